# baseline (speedup 1.0000x reference)
import functools

import jax
import jax.numpy as jnp
from jax import lax
from jax.experimental import pallas as pl
from jax.experimental.pallas import tpu as pltpu
from jax.experimental.pallas import tpu_sc as plsc

N = 10000
D = 128
G = 64
C = 16

NC = 2
NS = 16
NW = NC * NS
K = 128

N_ACC = 10240
RPT = N_ACC // NS

_f32 = jnp.float32


def _silu(v):
    return v * jax.nn.sigmoid(v)


NB = 4


def _sc_spmm(table, srcp, dstp, F, ept):
    ch = ept // K
    nb = 4
    nr = 2 * nb
    ni = ch // nr
    mesh = plsc.VectorSubcoreMesh(core_axis_name="c", subcore_axis_name="s")

    @functools.partial(
        pl.kernel,
        out_type=jax.ShapeDtypeStruct((NC * N_ACC, F), _f32),
        mesh=mesh,
        scratch_types=(
            [pltpu.VMEM((ept,), jnp.int32), pltpu.VMEM((ept,), jnp.int32)]
            + [pltpu.VMEM((K,), jnp.int32) for _ in range(nr)]
            + [pltpu.VMEM((K, F), _f32) for _ in range(nr)]
            + [pltpu.VMEM_SHARED((N_ACC, F), _f32)]
            + [pltpu.SemaphoreType.DMA for _ in range(2 * nr + 2)]
        ),
        compiler_params=pltpu.CompilerParams(use_tc_tiling_on_sc=False),
    )
    def k(table_hbm, src_hbm, dst_hbm, out_hbm, *sc):
        sidx_all, didx_all = sc[0], sc[1]
        didx = list(sc[2:2 + nr])
        rows = list(sc[2 + nr:2 + 2 * nr])
        acc = sc[2 + 2 * nr]
        gsem = list(sc[3 + 2 * nr:3 + 3 * nr])
        ssem = list(sc[3 + 3 * nr:3 + 4 * nr])
        isem0, isem1 = sc[3 + 4 * nr], sc[4 + 4 * nr]
        c = lax.axis_index("c")
        s = lax.axis_index("s")
        wid = s * NC + c
        base_e = wid * ept
        z16 = jnp.zeros((16,), _f32)

        pltpu.async_copy(src_hbm.at[pl.ds(base_e, ept)], sidx_all, isem0)
        pltpu.async_copy(dst_hbm.at[pl.ds(base_e, ept)], didx_all, isem1)

        def zrow(i, carry):
            for j in range(F // 16):
                rows[0][i, pl.ds(j * 16, 16)] = z16
            return carry

        lax.fori_loop(0, K, zrow, 0)

        def zcopy(p, carry):
            pltpu.sync_copy(rows[0], acc.at[pl.ds(s * RPT + p * K, K)])
            return carry

        lax.fori_loop(0, RPT // K, zcopy, 0)
        pltpu.make_async_copy(src_hbm.at[pl.ds(base_e, ept)], sidx_all, isem0).wait()
        pltpu.make_async_copy(dst_hbm.at[pl.ds(base_e, ept)], didx_all, isem1).wait()
        plsc.subcore_barrier()

        for b in range(nb):
            pltpu.async_copy(table_hbm.at[sidx_all.at[pl.ds(b * K, K)]],
                             rows[b], gsem[b])

        def outer(i, carry):
            for b in range(nr):
                g = i * nr + b
                off = g * K
                pltpu.make_async_copy(
                    table_hbm.at[sidx_all.at[pl.ds(0, K)]],
                    rows[b], gsem[b]).wait()
                for j in range(K // 16):
                    didx[b][pl.ds(j * 16, 16)] = didx_all[pl.ds(off + j * 16, 16)]
                pltpu.async_copy(rows[b], acc.at[didx[b]], ssem[b], add=True)
                pb = (b + nb) % nr

                @pl.when(g + nb < ch)
                def _():
                    @pl.when(g >= nb)
                    def _():
                        pltpu.make_async_copy(rows[pb], acc.at[didx[pb]],
                                              ssem[pb]).wait()

                    pltpu.async_copy(
                        table_hbm.at[sidx_all.at[pl.ds(off + nb * K, K)]],
                        rows[pb], gsem[pb])
            return carry

        lax.fori_loop(0, ni, outer, 0)
        for b in range(nr):
            pltpu.make_async_copy(rows[b], acc.at[didx[b]], ssem[b]).wait()
        plsc.subcore_barrier()
        pltpu.sync_copy(acc.at[pl.ds(s * RPT, RPT)],
                        out_hbm.at[pl.ds(c * N_ACC + s * RPT, RPT)])

    return k(table, srcp, dstp)


def _sc_hist(dstp, ept):
    F = 16
    ch = ept // K
    mesh = plsc.VectorSubcoreMesh(core_axis_name="c", subcore_axis_name="s")

    @functools.partial(
        pl.kernel,
        out_type=jax.ShapeDtypeStruct((NC * N_ACC, F), _f32),
        mesh=mesh,
        scratch_types=(
            [pltpu.VMEM((ept,), jnp.int32)]
            + [pltpu.VMEM((K,), jnp.int32) for _ in range(NB)]
            + [pltpu.VMEM((K, F), _f32),
               pltpu.VMEM((128, F), _f32),
               pltpu.VMEM_SHARED((N_ACC, F), _f32)]
            + [pltpu.SemaphoreType.DMA for _ in range(NB + 1)]
        ),
        compiler_params=pltpu.CompilerParams(use_tc_tiling_on_sc=False),
    )
    def k(dst_hbm, out_hbm, didx_all, d0, d1, d2, d3, rows, zbuf, acc,
          s0, s1, s2, s3, isem):
        didx = [d0, d1, d2, d3]
        ssem = [s0, s1, s2, s3]
        ni = ch // NB
        c = lax.axis_index("c")
        s = lax.axis_index("s")
        wid = s * NC + c
        z16 = jnp.zeros((16,), _f32)
        o16 = jnp.ones((16,), _f32)

        pltpu.async_copy(dst_hbm.at[pl.ds(wid * ept, ept)], didx_all, isem)

        def zrow(i, carry):
            zbuf[i, pl.ds(0, 16)] = z16
            return carry

        def orow(i, carry):
            rows[i, pl.ds(0, 16)] = o16
            return carry

        lax.fori_loop(0, 128, zrow, 0)
        lax.fori_loop(0, K, orow, 0)

        def zcopy(p, carry):
            pltpu.sync_copy(zbuf, acc.at[pl.ds(s * RPT + p * 128, 128)])
            return carry

        lax.fori_loop(0, RPT // 128, zcopy, 0)
        pltpu.make_async_copy(dst_hbm.at[pl.ds(wid * ept, ept)],
                              didx_all, isem).wait()
        plsc.subcore_barrier()

        def outer(i, carry):
            for b in range(NB):
                off = (i * NB + b) * K

                @pl.when(i > 0)
                def _():
                    pltpu.make_async_copy(rows, acc.at[didx[b]], ssem[b]).wait()

                for j in range(K // 16):
                    didx[b][pl.ds(j * 16, 16)] = didx_all[pl.ds(off + j * 16, 16)]
                pltpu.async_copy(rows, acc.at[didx[b]], ssem[b], add=True)
            return carry

        lax.fori_loop(0, ni, outer, 0)
        for b in range(NB):
            pltpu.make_async_copy(rows, acc.at[didx[b]], ssem[b]).wait()
        plsc.subcore_barrier()
        pltpu.sync_copy(acc.at[pl.ds(s * RPT, RPT)],
                        out_hbm.at[pl.ds(c * N_ACC + s * RPT, RPT)])

    return k(dstp)


def _tc_stage1(x, w1t, wr1t, br1, degp):
    def body(x_ref, w1_ref, wr1_ref, br1_ref, degp_ref,
             hh1_ref, h1_ref, xr1_ref, dinv_ref):
        deg = degp_ref[0] + degp_ref[1] + 1.0
        dinv = lax.rsqrt(deg)
        xv = x_ref[...]
        h1 = jnp.dot(xv, w1_ref[...], preferred_element_type=_f32)
        xr1 = _silu(jnp.dot(xv, wr1_ref[...], preferred_element_type=_f32)
                    + br1_ref[...])
        h1_ref[...] = h1
        hh1_ref[...] = h1 * dinv
        xr1_ref[...] = xr1
        dinv_ref[...] = dinv

    return pl.pallas_call(
        body,
        out_shape=[
            jax.ShapeDtypeStruct((N, 64), _f32),
            jax.ShapeDtypeStruct((N, 64), _f32),
            jax.ShapeDtypeStruct((N, 64), _f32),
            jax.ShapeDtypeStruct((N, 1), _f32),
        ],
    )(x, w1t, wr1t, br1, degp)


def _tc_stage2(p1, h1, xr1, dinv, b1, a1, w2t, wr2t, br2):
    def body(p_ref, h1_ref, xr1_ref, dinv_ref, b1_ref, a1_ref,
             w2_ref, wr2_ref, br2_ref, hh2_ref, h2_ref, xr2_ref):
        dinv = dinv_ref[...]
        agg = p_ref[0] + p_ref[1]
        conv1 = dinv * agg + (dinv * dinv) * h1_ref[...] + b1_ref[...]
        h = _silu(conv1) + a1_ref[0, 0] * xr1_ref[...]
        h2 = jnp.dot(h, w2_ref[...], preferred_element_type=_f32)
        xr2 = _silu(jnp.dot(h, wr2_ref[...], preferred_element_type=_f32)
                    + br2_ref[...])
        hh2_ref[...] = h2 * dinv
        h2_ref[...] = h2
        xr2_ref[...] = xr2

    return pl.pallas_call(
        body,
        out_shape=[
            jax.ShapeDtypeStruct((N, 16), _f32),
            jax.ShapeDtypeStruct((N, 16), _f32),
            jax.ShapeDtypeStruct((N, 16), _f32),
        ],
    )(p1, h1, xr1, dinv, b1, a1, w2t, wr2t, br2)


def _tc_stage3(p2, h2, xr2, dinv, b2, a2, batch_row, scalars, heads,
               wf1, bf1, wf2t, bf2):
    def body(p_ref, h2_ref, xr2_ref, dinv_ref, b2_ref, a2_ref, batch_ref,
             tol_ref, cst_ref, tim_ref, qty_ref,
             wt1_ref, bt1_ref, wt2_ref, bt2_ref,
             wc1_ref, bc1_ref, wc2_ref, bc2_ref,
             wm1_ref, bm1_ref, wm2_ref, bm2_ref,
             wq1_ref, bq1_ref, wq2_ref, bq2_ref,
             wf1_ref, bf1_ref, wf2_ref, bf2_ref, out_ref):
        dinv = dinv_ref[...]
        agg = p_ref[0] + p_ref[1]
        z = (dinv * agg + (dinv * dinv) * h2_ref[...] + b2_ref[...]
             + a2_ref[0, 0] * xr2_ref[...])
        gids = lax.broadcasted_iota(jnp.int32, (G, N), 0)
        mask = jnp.where(batch_ref[...] == gids, 1.0, 0.0).astype(_f32)
        sums = jnp.dot(mask, z, preferred_element_type=_f32)
        cnt = jnp.sum(mask, axis=1, keepdims=True)
        ge = sums / jnp.maximum(cnt, 1.0)

        def head(v_ref, wa_ref, ba_ref, wb_ref, bb_ref):
            hmid = _silu(v_ref[0, 0] * wa_ref[...] + ba_ref[...])
            return (jnp.dot(hmid, wb_ref[...], preferred_element_type=_f32)
                    + bb_ref[...])

        tol = jnp.broadcast_to(head(tol_ref, wt1_ref, bt1_ref, wt2_ref, bt2_ref), (G, C))
        cst = jnp.broadcast_to(head(cst_ref, wc1_ref, bc1_ref, wc2_ref, bc2_ref), (G, C))
        tim = jnp.broadcast_to(head(tim_ref, wm1_ref, bm1_ref, wm2_ref, bm2_ref), (G, C))
        qty = jnp.broadcast_to(head(qty_ref, wq1_ref, bq1_ref, wq2_ref, bq2_ref), (G, C))
        comb = jnp.concatenate([ge, tol, cst, tim, qty], axis=1)
        o = _silu(jnp.dot(comb, wf1_ref[...], preferred_element_type=_f32)
                  + bf1_ref[...])
        out_ref[...] = (jnp.dot(o, wf2_ref[...], preferred_element_type=_f32)
                        + bf2_ref[...])

    args = ([p2, h2, xr2, dinv, b2, a2, batch_row] + scalars + heads
            + [wf1, bf1, wf2t, bf2])
    return pl.pallas_call(
        body,
        out_shape=jax.ShapeDtypeStruct((G, C), _f32),
    )(*args)


def kernel(x, edge_index, batch, tolerance, cost, time, quantity,
           W1, b1, W2, b2, Wr1, br1, Wr2, br2, alpha1, alpha2,
           Wt1, bt1, Wt2, bt2, Wc1, bc1, Wc2, bc2, Wm1, bm1, Wm2, bm2,
           Wq1, bq1, Wq2, bq2, Wf1, bf1, Wf2, bf2):
    E = edge_index.shape[1]
    ept = -(-E // (NW * K * 8)) * K * 8
    e_pad = NW * ept
    pad = e_pad - E
    src = jnp.concatenate([edge_index[0], jnp.zeros((pad,), jnp.int32)])
    dst = jnp.concatenate([edge_index[1], jnp.full((pad,), N, jnp.int32)])

    degp_flat = _sc_hist(dst, ept)
    degp = degp_flat.reshape(NC, N_ACC, 16)[:, :N, 0:1]

    hh1, h1, xr1, dinv = _tc_stage1(
        x, W1.T, Wr1.T, br1.reshape(1, 64), degp)

    p1 = _sc_spmm(hh1, src, dst, 64, ept).reshape(NC, N_ACC, 64)[:, :N, :]

    hh2, h2, xr2 = _tc_stage2(
        p1, h1, xr1, dinv, b1.reshape(1, 64),
        alpha1.reshape(1, 1), W2.T, Wr2.T, br2.reshape(1, 16))

    p2 = _sc_spmm(hh2, src, dst, 16, ept).reshape(NC, N_ACC, 16)[:, :N, :]

    scalars = [tolerance, cost, time, quantity]
    heads = [Wt1.reshape(1, 8), bt1.reshape(1, 8), Wt2.T, bt2.reshape(1, 16),
             Wc1.reshape(1, 8), bc1.reshape(1, 8), Wc2.T, bc2.reshape(1, 16),
             Wm1.reshape(1, 8), bm1.reshape(1, 8), Wm2.T, bm2.reshape(1, 16),
             Wq1.reshape(1, 8), bq1.reshape(1, 8), Wq2.T, bq2.reshape(1, 16)]
    out = _tc_stage3(
        p2, h2, xr2, dinv, b2.reshape(1, 16), alpha2.reshape(1, 1),
        batch.reshape(1, N), scalars, heads, Wf1.T, bf1.reshape(1, 80),
        Wf2.T, bf2.reshape(1, 16))
    return out


# speedup vs baseline: 1.7618x; 1.7618x over previous
import functools

import jax
import jax.numpy as jnp
from jax import lax
from jax.experimental import pallas as pl
from jax.experimental.pallas import tpu as pltpu
from jax.experimental.pallas import tpu_sc as plsc

N = 10000
D = 128
G = 64
C = 16

NC = 2
NS = 16
NW = NC * NS
K = 128

N_ACC = 10240
RPT = N_ACC // NS
N_TB = 10240
TBR = N_TB // NS

_f32 = jnp.float32


def _silu(v):
    return v * jax.nn.sigmoid(v)


NB = 4


def _sc_spmm(table, srcp, dstp, F, ept):
    ch = ept // K
    mesh = plsc.VectorSubcoreMesh(core_axis_name="c", subcore_axis_name="s")

    @functools.partial(
        pl.kernel,
        out_type=jax.ShapeDtypeStruct((NC * N_ACC, F), _f32),
        mesh=mesh,
        scratch_types=(
            [pltpu.VMEM((ept,), jnp.int32), pltpu.VMEM((ept,), jnp.int32)]
            + [pltpu.VMEM((K,), jnp.int32) for _ in range(2)]
            + [pltpu.VMEM((K, F), _f32) for _ in range(2)]
            + [pltpu.VMEM_SHARED((N_TB, F), _f32),
               pltpu.VMEM_SHARED((N_ACC, F), _f32)]
            + [pltpu.SemaphoreType.DMA for _ in range(7)]
        ),
        compiler_params=pltpu.CompilerParams(use_tc_tiling_on_sc=False),
    )
    def k(table_hbm, src_hbm, dst_hbm, out_hbm,
          sidx_all, didx_all, d0, d1, r0, r1, tbl, acc,
          isem0, isem1, tsem, g0, g1, s0, s1):
        didx = [d0, d1]
        rows = [r0, r1]
        gsem = [g0, g1]
        ssem = [s0, s1]
        c = lax.axis_index("c")
        s = lax.axis_index("s")
        wid = s * NC + c
        base_e = wid * ept
        z16 = jnp.zeros((16,), _f32)

        pltpu.async_copy(src_hbm.at[pl.ds(base_e, ept)], sidx_all, isem0)
        pltpu.async_copy(dst_hbm.at[pl.ds(base_e, ept)], didx_all, isem1)
        pltpu.async_copy(table_hbm.at[pl.ds(s * TBR, TBR)],
                         tbl.at[pl.ds(s * TBR, TBR)], tsem)

        def zrow(i, carry):
            for j in range(F // 16):
                rows[0][i, pl.ds(j * 16, 16)] = z16
            return carry

        lax.fori_loop(0, K, zrow, 0)

        def zcopy(p, carry):
            pltpu.sync_copy(rows[0], acc.at[pl.ds(s * RPT + p * K, K)])
            return carry

        lax.fori_loop(0, RPT // K, zcopy, 0)
        pltpu.make_async_copy(src_hbm.at[pl.ds(base_e, ept)], sidx_all, isem0).wait()
        pltpu.make_async_copy(dst_hbm.at[pl.ds(base_e, ept)], didx_all, isem1).wait()
        pltpu.make_async_copy(table_hbm.at[pl.ds(s * TBR, TBR)],
                              tbl.at[pl.ds(s * TBR, TBR)], tsem).wait()
        plsc.subcore_barrier()

        pltpu.async_copy(tbl.at[sidx_all.at[pl.ds(0, K)]], rows[0], gsem[0])

        def outer(i, carry):
            for b in range(2):
                g = 2 * i + b
                off = g * K
                ob = 1 - b
                pltpu.make_async_copy(
                    tbl.at[sidx_all.at[pl.ds(0, K)]],
                    rows[b], gsem[b]).wait()
                for j in range(K // 16):
                    didx[b][pl.ds(j * 16, 16)] = didx_all[pl.ds(off + j * 16, 16)]
                pltpu.async_copy(rows[b], acc.at[didx[b]], ssem[b], add=True)

                @pl.when(g + 1 < ch)
                def _():
                    @pl.when(g >= 1)
                    def _():
                        pltpu.make_async_copy(rows[ob], acc.at[didx[ob]],
                                              ssem[ob]).wait()

                    pltpu.async_copy(
                        tbl.at[sidx_all.at[pl.ds(off + K, K)]],
                        rows[ob], gsem[ob])
            return carry

        lax.fori_loop(0, ch // 2, outer, 0)
        for b in range(2):
            pltpu.make_async_copy(rows[b], acc.at[didx[b]], ssem[b]).wait()
        plsc.subcore_barrier()
        pltpu.sync_copy(acc.at[pl.ds(s * RPT, RPT)],
                        out_hbm.at[pl.ds(c * N_ACC + s * RPT, RPT)])

    return k(table, srcp, dstp)


def _sc_hist(dstp, ept):
    F = 16
    ch = ept // K
    mesh = plsc.VectorSubcoreMesh(core_axis_name="c", subcore_axis_name="s")

    @functools.partial(
        pl.kernel,
        out_type=jax.ShapeDtypeStruct((NC * N_ACC, F), _f32),
        mesh=mesh,
        scratch_types=(
            [pltpu.VMEM((ept,), jnp.int32)]
            + [pltpu.VMEM((K,), jnp.int32) for _ in range(NB)]
            + [pltpu.VMEM((K, F), _f32),
               pltpu.VMEM((128, F), _f32),
               pltpu.VMEM_SHARED((N_ACC, F), _f32)]
            + [pltpu.SemaphoreType.DMA for _ in range(NB + 1)]
        ),
        compiler_params=pltpu.CompilerParams(use_tc_tiling_on_sc=False),
    )
    def k(dst_hbm, out_hbm, didx_all, d0, d1, d2, d3, rows, zbuf, acc,
          s0, s1, s2, s3, isem):
        didx = [d0, d1, d2, d3]
        ssem = [s0, s1, s2, s3]
        ni = ch // NB
        c = lax.axis_index("c")
        s = lax.axis_index("s")
        wid = s * NC + c
        z16 = jnp.zeros((16,), _f32)
        o16 = jnp.ones((16,), _f32)

        pltpu.async_copy(dst_hbm.at[pl.ds(wid * ept, ept)], didx_all, isem)

        def zrow(i, carry):
            zbuf[i, pl.ds(0, 16)] = z16
            return carry

        def orow(i, carry):
            rows[i, pl.ds(0, 16)] = o16
            return carry

        lax.fori_loop(0, 128, zrow, 0)
        lax.fori_loop(0, K, orow, 0)

        def zcopy(p, carry):
            pltpu.sync_copy(zbuf, acc.at[pl.ds(s * RPT + p * 128, 128)])
            return carry

        lax.fori_loop(0, RPT // 128, zcopy, 0)
        pltpu.make_async_copy(dst_hbm.at[pl.ds(wid * ept, ept)],
                              didx_all, isem).wait()
        plsc.subcore_barrier()

        def outer(i, carry):
            for b in range(NB):
                off = (i * NB + b) * K

                @pl.when(i > 0)
                def _():
                    pltpu.make_async_copy(rows, acc.at[didx[b]], ssem[b]).wait()

                for j in range(K // 16):
                    didx[b][pl.ds(j * 16, 16)] = didx_all[pl.ds(off + j * 16, 16)]
                pltpu.async_copy(rows, acc.at[didx[b]], ssem[b], add=True)
            return carry

        lax.fori_loop(0, ni, outer, 0)
        for b in range(NB):
            pltpu.make_async_copy(rows, acc.at[didx[b]], ssem[b]).wait()
        plsc.subcore_barrier()
        pltpu.sync_copy(acc.at[pl.ds(s * RPT, RPT)],
                        out_hbm.at[pl.ds(c * N_ACC + s * RPT, RPT)])

    return k(dstp)


def _tc_stage1(x, w1t, wr1t, br1, degp):
    def body(x_ref, w1_ref, wr1_ref, br1_ref, degp_ref,
             hh1_ref, h1_ref, xr1_ref, dinv_ref):
        deg = degp_ref[0] + degp_ref[1] + 1.0
        dinv = lax.rsqrt(deg)
        xv = x_ref[...]
        h1 = jnp.dot(xv, w1_ref[...], preferred_element_type=_f32)
        xr1 = _silu(jnp.dot(xv, wr1_ref[...], preferred_element_type=_f32)
                    + br1_ref[...])
        h1_ref[...] = h1
        hh1_ref[pl.ds(0, N), :] = h1 * dinv
        xr1_ref[...] = xr1
        dinv_ref[...] = dinv

    return pl.pallas_call(
        body,
        out_shape=[
            jax.ShapeDtypeStruct((N_TB, 64), _f32),
            jax.ShapeDtypeStruct((N, 64), _f32),
            jax.ShapeDtypeStruct((N, 64), _f32),
            jax.ShapeDtypeStruct((N, 1), _f32),
        ],
    )(x, w1t, wr1t, br1, degp)


def _tc_stage2(p1, h1, xr1, dinv, b1, a1, w2t, wr2t, br2):
    def body(p_ref, h1_ref, xr1_ref, dinv_ref, b1_ref, a1_ref,
             w2_ref, wr2_ref, br2_ref, hh2_ref, h2_ref, xr2_ref):
        dinv = dinv_ref[...]
        agg = p_ref[0] + p_ref[1]
        conv1 = dinv * agg + (dinv * dinv) * h1_ref[...] + b1_ref[...]
        h = _silu(conv1) + a1_ref[0, 0] * xr1_ref[...]
        h2 = jnp.dot(h, w2_ref[...], preferred_element_type=_f32)
        xr2 = _silu(jnp.dot(h, wr2_ref[...], preferred_element_type=_f32)
                    + br2_ref[...])
        hh2_ref[pl.ds(0, N), :] = h2 * dinv
        h2_ref[...] = h2
        xr2_ref[...] = xr2

    return pl.pallas_call(
        body,
        out_shape=[
            jax.ShapeDtypeStruct((N_TB, 16), _f32),
            jax.ShapeDtypeStruct((N, 16), _f32),
            jax.ShapeDtypeStruct((N, 16), _f32),
        ],
    )(p1, h1, xr1, dinv, b1, a1, w2t, wr2t, br2)


def _tc_stage3(p2, h2, xr2, dinv, b2, a2, batch_row, scalars, heads,
               wf1, bf1, wf2t, bf2):
    def body(p_ref, h2_ref, xr2_ref, dinv_ref, b2_ref, a2_ref, batch_ref,
             tol_ref, cst_ref, tim_ref, qty_ref,
             wt1_ref, bt1_ref, wt2_ref, bt2_ref,
             wc1_ref, bc1_ref, wc2_ref, bc2_ref,
             wm1_ref, bm1_ref, wm2_ref, bm2_ref,
             wq1_ref, bq1_ref, wq2_ref, bq2_ref,
             wf1_ref, bf1_ref, wf2_ref, bf2_ref, out_ref):
        dinv = dinv_ref[...]
        agg = p_ref[0] + p_ref[1]
        z = (dinv * agg + (dinv * dinv) * h2_ref[...] + b2_ref[...]
             + a2_ref[0, 0] * xr2_ref[...])
        gids = lax.broadcasted_iota(jnp.int32, (G, N), 0)
        mask = jnp.where(batch_ref[...] == gids, 1.0, 0.0).astype(_f32)
        sums = jnp.dot(mask, z, preferred_element_type=_f32)
        cnt = jnp.sum(mask, axis=1, keepdims=True)
        ge = sums / jnp.maximum(cnt, 1.0)

        def head(v_ref, wa_ref, ba_ref, wb_ref, bb_ref):
            hmid = _silu(v_ref[0, 0] * wa_ref[...] + ba_ref[...])
            return (jnp.dot(hmid, wb_ref[...], preferred_element_type=_f32)
                    + bb_ref[...])

        tol = jnp.broadcast_to(head(tol_ref, wt1_ref, bt1_ref, wt2_ref, bt2_ref), (G, C))
        cst = jnp.broadcast_to(head(cst_ref, wc1_ref, bc1_ref, wc2_ref, bc2_ref), (G, C))
        tim = jnp.broadcast_to(head(tim_ref, wm1_ref, bm1_ref, wm2_ref, bm2_ref), (G, C))
        qty = jnp.broadcast_to(head(qty_ref, wq1_ref, bq1_ref, wq2_ref, bq2_ref), (G, C))
        comb = jnp.concatenate([ge, tol, cst, tim, qty], axis=1)
        o = _silu(jnp.dot(comb, wf1_ref[...], preferred_element_type=_f32)
                  + bf1_ref[...])
        out_ref[...] = (jnp.dot(o, wf2_ref[...], preferred_element_type=_f32)
                        + bf2_ref[...])

    args = ([p2, h2, xr2, dinv, b2, a2, batch_row] + scalars + heads
            + [wf1, bf1, wf2t, bf2])
    return pl.pallas_call(
        body,
        out_shape=jax.ShapeDtypeStruct((G, C), _f32),
    )(*args)


def kernel(x, edge_index, batch, tolerance, cost, time, quantity,
           W1, b1, W2, b2, Wr1, br1, Wr2, br2, alpha1, alpha2,
           Wt1, bt1, Wt2, bt2, Wc1, bc1, Wc2, bc2, Wm1, bm1, Wm2, bm2,
           Wq1, bq1, Wq2, bq2, Wf1, bf1, Wf2, bf2):
    E = edge_index.shape[1]
    ept = -(-E // (NW * K * 8)) * K * 8
    e_pad = NW * ept
    pad = e_pad - E
    src = jnp.concatenate([edge_index[0], jnp.zeros((pad,), jnp.int32)])
    dst = jnp.concatenate([edge_index[1], jnp.full((pad,), N, jnp.int32)])

    degp_flat = _sc_hist(dst, ept)
    degp = degp_flat.reshape(NC, N_ACC, 16)[:, :N, 0:1]

    hh1, h1, xr1, dinv = _tc_stage1(
        x, W1.T, Wr1.T, br1.reshape(1, 64), degp)

    p1 = _sc_spmm(hh1, src, dst, 64, ept).reshape(NC, N_ACC, 64)[:, :N, :]

    hh2, h2, xr2 = _tc_stage2(
        p1, h1, xr1, dinv, b1.reshape(1, 64),
        alpha1.reshape(1, 1), W2.T, Wr2.T, br2.reshape(1, 16))

    p2 = _sc_spmm(hh2, src, dst, 16, ept).reshape(NC, N_ACC, 16)[:, :N, :]

    scalars = [tolerance, cost, time, quantity]
    heads = [Wt1.reshape(1, 8), bt1.reshape(1, 8), Wt2.T, bt2.reshape(1, 16),
             Wc1.reshape(1, 8), bc1.reshape(1, 8), Wc2.T, bc2.reshape(1, 16),
             Wm1.reshape(1, 8), bm1.reshape(1, 8), Wm2.T, bm2.reshape(1, 16),
             Wq1.reshape(1, 8), bq1.reshape(1, 8), Wq2.T, bq2.reshape(1, 16)]
    out = _tc_stage3(
        p2, h2, xr2, dinv, b2.reshape(1, 16), alpha2.reshape(1, 1),
        batch.reshape(1, N), scalars, heads, Wf1.T, bf1.reshape(1, 80),
        Wf2.T, bf2.reshape(1, 16))
    return out


# speedup vs baseline: 1.7932x; 1.0179x over previous
import functools

import jax
import jax.numpy as jnp
from jax import lax
from jax.experimental import pallas as pl
from jax.experimental.pallas import tpu as pltpu
from jax.experimental.pallas import tpu_sc as plsc

N = 10000
D = 128
G = 64
C = 16

NC = 2
NS = 16
NW = NC * NS
K = 128

N_ACC = 10240
RPT = N_ACC // NS
N_TB = 10240
TBR = N_TB // NS

_f32 = jnp.float32


def _silu(v):
    return v * jax.nn.sigmoid(v)


NB = 4


def _sc_spmm(table, srcp, dstp, F, ept):
    ch = ept // K
    mesh = plsc.VectorSubcoreMesh(core_axis_name="c", subcore_axis_name="s")

    @functools.partial(
        pl.kernel,
        out_type=jax.ShapeDtypeStruct((NC * N_ACC, F), _f32),
        mesh=mesh,
        scratch_types=(
            [pltpu.VMEM((ept,), jnp.int32), pltpu.VMEM((ept,), jnp.int32)]
            + [pltpu.VMEM((K,), jnp.int32) for _ in range(2)]
            + [pltpu.VMEM((K, F), _f32) for _ in range(2)]
            + [pltpu.VMEM_SHARED((N_TB, F), _f32),
               pltpu.VMEM_SHARED((N_ACC, F), _f32)]
            + [pltpu.SemaphoreType.DMA for _ in range(7)]
        ),
        compiler_params=pltpu.CompilerParams(use_tc_tiling_on_sc=False),
    )
    def k(table_hbm, src_hbm, dst_hbm, out_hbm,
          sidx_all, didx_all, d0, d1, r0, r1, tbl, acc,
          isem0, isem1, tsem, g0, g1, s0, s1):
        didx = [d0, d1]
        rows = [r0, r1]
        gsem = [g0, g1]
        ssem = [s0, s1]
        c = lax.axis_index("c")
        s = lax.axis_index("s")
        wid = s * NC + c
        base_e = wid * ept
        z16 = jnp.zeros((16,), _f32)

        pltpu.async_copy(src_hbm.at[pl.ds(base_e, ept)], sidx_all, isem0)
        pltpu.async_copy(dst_hbm.at[pl.ds(base_e, ept)], didx_all, isem1)
        pltpu.async_copy(table_hbm.at[pl.ds(s * TBR, TBR)],
                         tbl.at[pl.ds(s * TBR, TBR)], tsem)

        def zrow(i, carry):
            for j in range(F // 16):
                rows[0][i, pl.ds(j * 16, 16)] = z16
            return carry

        lax.fori_loop(0, K, zrow, 0)

        def zcopy(p, carry):
            pltpu.sync_copy(rows[0], acc.at[pl.ds(s * RPT + p * K, K)])
            return carry

        lax.fori_loop(0, RPT // K, zcopy, 0)
        pltpu.make_async_copy(src_hbm.at[pl.ds(base_e, ept)], sidx_all, isem0).wait()
        pltpu.make_async_copy(dst_hbm.at[pl.ds(base_e, ept)], didx_all, isem1).wait()
        pltpu.make_async_copy(table_hbm.at[pl.ds(s * TBR, TBR)],
                              tbl.at[pl.ds(s * TBR, TBR)], tsem).wait()
        plsc.subcore_barrier()

        pltpu.async_copy(tbl.at[sidx_all.at[pl.ds(0, K)]], rows[0], gsem[0])

        def outer(i, carry):
            for b in range(2):
                g = 2 * i + b
                off = g * K
                ob = 1 - b
                pltpu.make_async_copy(
                    tbl.at[sidx_all.at[pl.ds(0, K)]],
                    rows[b], gsem[b]).wait()
                for j in range(K // 16):
                    didx[b][pl.ds(j * 16, 16)] = didx_all[pl.ds(off + j * 16, 16)]
                pltpu.async_copy(rows[b], acc.at[didx[b]], ssem[b], add=True)

                @pl.when(g + 1 < ch)
                def _():
                    @pl.when(g >= 1)
                    def _():
                        pltpu.make_async_copy(rows[ob], acc.at[didx[ob]],
                                              ssem[ob]).wait()

                    pltpu.async_copy(
                        tbl.at[sidx_all.at[pl.ds(off + K, K)]],
                        rows[ob], gsem[ob])
            return carry

        lax.fori_loop(0, ch // 2, outer, 0)
        for b in range(2):
            pltpu.make_async_copy(rows[b], acc.at[didx[b]], ssem[b]).wait()
        plsc.subcore_barrier()
        pltpu.sync_copy(acc.at[pl.ds(s * RPT, RPT)],
                        out_hbm.at[pl.ds(c * N_ACC + s * RPT, RPT)])

    return k(table, srcp, dstp)


def _sc_hist(dstp, ept):
    F = 16
    ch = ept // K
    mesh = plsc.VectorSubcoreMesh(core_axis_name="c", subcore_axis_name="s")

    @functools.partial(
        pl.kernel,
        out_type=jax.ShapeDtypeStruct((NC * N_ACC, F), _f32),
        mesh=mesh,
        scratch_types=(
            [pltpu.VMEM((ept,), jnp.int32)]
            + [pltpu.VMEM((K,), jnp.int32) for _ in range(NB)]
            + [pltpu.VMEM((K, F), _f32),
               pltpu.VMEM((128, F), _f32),
               pltpu.VMEM_SHARED((N_ACC, F), _f32)]
            + [pltpu.SemaphoreType.DMA for _ in range(NB + 1)]
        ),
        compiler_params=pltpu.CompilerParams(use_tc_tiling_on_sc=False),
    )
    def k(dst_hbm, out_hbm, didx_all, d0, d1, d2, d3, rows, zbuf, acc,
          s0, s1, s2, s3, isem):
        didx = [d0, d1, d2, d3]
        ssem = [s0, s1, s2, s3]
        ni = ch // NB
        c = lax.axis_index("c")
        s = lax.axis_index("s")
        wid = s * NC + c
        z16 = jnp.zeros((16,), _f32)
        o16 = jnp.ones((16,), _f32)

        pltpu.async_copy(dst_hbm.at[pl.ds(wid * ept, ept)], didx_all, isem)

        def zrow(i, carry):
            zbuf[i, pl.ds(0, 16)] = z16
            return carry

        def orow(i, carry):
            rows[i, pl.ds(0, 16)] = o16
            return carry

        lax.fori_loop(0, 128, zrow, 0)
        lax.fori_loop(0, K, orow, 0)

        def zcopy(p, carry):
            pltpu.sync_copy(zbuf, acc.at[pl.ds(s * RPT + p * 128, 128)])
            return carry

        lax.fori_loop(0, RPT // 128, zcopy, 0)
        pltpu.make_async_copy(dst_hbm.at[pl.ds(wid * ept, ept)],
                              didx_all, isem).wait()
        plsc.subcore_barrier()

        def outer(i, carry):
            for b in range(NB):
                off = (i * NB + b) * K

                @pl.when(i > 0)
                def _():
                    pltpu.make_async_copy(rows, acc.at[didx[b]], ssem[b]).wait()

                for j in range(K // 16):
                    didx[b][pl.ds(j * 16, 16)] = didx_all[pl.ds(off + j * 16, 16)]
                pltpu.async_copy(rows, acc.at[didx[b]], ssem[b], add=True)
            return carry

        lax.fori_loop(0, ni, outer, 0)
        for b in range(NB):
            pltpu.make_async_copy(rows, acc.at[didx[b]], ssem[b]).wait()
        plsc.subcore_barrier()
        pltpu.sync_copy(acc.at[pl.ds(s * RPT, RPT)],
                        out_hbm.at[pl.ds(c * N_ACC + s * RPT, RPT)])

    return k(dstp)


def _tc_stage1a(x, w1t, wr1t, br1):
    def body(x_ref, w1_ref, wr1_ref, br1_ref, h1_ref, xr1_ref):
        xv = x_ref[...]
        h1_ref[...] = jnp.dot(xv, w1_ref[...], preferred_element_type=_f32)
        xr1_ref[...] = _silu(
            jnp.dot(xv, wr1_ref[...], preferred_element_type=_f32)
            + br1_ref[...])

    return pl.pallas_call(
        body,
        out_shape=[
            jax.ShapeDtypeStruct((N, 64), _f32),
            jax.ShapeDtypeStruct((N, 64), _f32),
        ],
    )(x, w1t, wr1t, br1)


def _tc_stage1b(h1, degp):
    def body(h1_ref, degp_ref, hh1_ref, dinv_ref):
        deg = degp_ref[0] + degp_ref[1] + 1.0
        dinv = lax.rsqrt(deg)
        hh1_ref[pl.ds(0, N), :] = h1_ref[...] * dinv
        dinv_ref[...] = dinv

    return pl.pallas_call(
        body,
        out_shape=[
            jax.ShapeDtypeStruct((N_TB, 64), _f32),
            jax.ShapeDtypeStruct((N, 1), _f32),
        ],
    )(h1, degp)


def _tc_stage2(p1, h1, xr1, dinv, b1, a1, w2t, wr2t, br2):
    def body(p_ref, h1_ref, xr1_ref, dinv_ref, b1_ref, a1_ref,
             w2_ref, wr2_ref, br2_ref, hh2_ref, h2_ref, xr2_ref):
        dinv = dinv_ref[...]
        agg = p_ref[0] + p_ref[1]
        conv1 = dinv * agg + (dinv * dinv) * h1_ref[...] + b1_ref[...]
        h = _silu(conv1) + a1_ref[0, 0] * xr1_ref[...]
        h2 = jnp.dot(h, w2_ref[...], preferred_element_type=_f32)
        xr2 = _silu(jnp.dot(h, wr2_ref[...], preferred_element_type=_f32)
                    + br2_ref[...])
        hh2_ref[pl.ds(0, N), :] = h2 * dinv
        h2_ref[...] = h2
        xr2_ref[...] = xr2

    return pl.pallas_call(
        body,
        out_shape=[
            jax.ShapeDtypeStruct((N_TB, 16), _f32),
            jax.ShapeDtypeStruct((N, 16), _f32),
            jax.ShapeDtypeStruct((N, 16), _f32),
        ],
    )(p1, h1, xr1, dinv, b1, a1, w2t, wr2t, br2)


def _tc_stage3(p2, h2, xr2, dinv, b2, a2, batch_row, scalars, heads,
               wf1, bf1, wf2t, bf2):
    def body(p_ref, h2_ref, xr2_ref, dinv_ref, b2_ref, a2_ref, batch_ref,
             tol_ref, cst_ref, tim_ref, qty_ref,
             wt1_ref, bt1_ref, wt2_ref, bt2_ref,
             wc1_ref, bc1_ref, wc2_ref, bc2_ref,
             wm1_ref, bm1_ref, wm2_ref, bm2_ref,
             wq1_ref, bq1_ref, wq2_ref, bq2_ref,
             wf1_ref, bf1_ref, wf2_ref, bf2_ref, out_ref):
        dinv = dinv_ref[...]
        agg = p_ref[0] + p_ref[1]
        z = (dinv * agg + (dinv * dinv) * h2_ref[...] + b2_ref[...]
             + a2_ref[0, 0] * xr2_ref[...])
        gids = lax.broadcasted_iota(jnp.int32, (G, N), 0)
        mask = jnp.where(batch_ref[...] == gids, 1.0, 0.0).astype(_f32)
        sums = jnp.dot(mask, z, preferred_element_type=_f32)
        cnt = jnp.sum(mask, axis=1, keepdims=True)
        ge = sums / jnp.maximum(cnt, 1.0)

        def head(v_ref, wa_ref, ba_ref, wb_ref, bb_ref):
            hmid = _silu(v_ref[0, 0] * wa_ref[...] + ba_ref[...])
            return (jnp.dot(hmid, wb_ref[...], preferred_element_type=_f32)
                    + bb_ref[...])

        tol = jnp.broadcast_to(head(tol_ref, wt1_ref, bt1_ref, wt2_ref, bt2_ref), (G, C))
        cst = jnp.broadcast_to(head(cst_ref, wc1_ref, bc1_ref, wc2_ref, bc2_ref), (G, C))
        tim = jnp.broadcast_to(head(tim_ref, wm1_ref, bm1_ref, wm2_ref, bm2_ref), (G, C))
        qty = jnp.broadcast_to(head(qty_ref, wq1_ref, bq1_ref, wq2_ref, bq2_ref), (G, C))
        comb = jnp.concatenate([ge, tol, cst, tim, qty], axis=1)
        o = _silu(jnp.dot(comb, wf1_ref[...], preferred_element_type=_f32)
                  + bf1_ref[...])
        out_ref[...] = (jnp.dot(o, wf2_ref[...], preferred_element_type=_f32)
                        + bf2_ref[...])

    args = ([p2, h2, xr2, dinv, b2, a2, batch_row] + scalars + heads
            + [wf1, bf1, wf2t, bf2])
    return pl.pallas_call(
        body,
        out_shape=jax.ShapeDtypeStruct((G, C), _f32),
    )(*args)


def kernel(x, edge_index, batch, tolerance, cost, time, quantity,
           W1, b1, W2, b2, Wr1, br1, Wr2, br2, alpha1, alpha2,
           Wt1, bt1, Wt2, bt2, Wc1, bc1, Wc2, bc2, Wm1, bm1, Wm2, bm2,
           Wq1, bq1, Wq2, bq2, Wf1, bf1, Wf2, bf2):
    E = edge_index.shape[1]
    ept = -(-E // (NW * K * 8)) * K * 8
    e_pad = NW * ept
    pad = e_pad - E
    src = jnp.concatenate([edge_index[0], jnp.zeros((pad,), jnp.int32)])
    dst = jnp.concatenate([edge_index[1], jnp.full((pad,), N, jnp.int32)])

    degp_flat = _sc_hist(dst, ept)
    degp = degp_flat.reshape(NC, N_ACC, 16)[:, :N, 0:1]

    h1, xr1 = _tc_stage1a(x, W1.T, Wr1.T, br1.reshape(1, 64))
    hh1, dinv = _tc_stage1b(h1, degp)

    p1 = _sc_spmm(hh1, src, dst, 64, ept).reshape(NC, N_ACC, 64)[:, :N, :]

    hh2, h2, xr2 = _tc_stage2(
        p1, h1, xr1, dinv, b1.reshape(1, 64),
        alpha1.reshape(1, 1), W2.T, Wr2.T, br2.reshape(1, 16))

    p2 = _sc_spmm(hh2, src, dst, 16, ept).reshape(NC, N_ACC, 16)[:, :N, :]

    scalars = [tolerance, cost, time, quantity]
    heads = [Wt1.reshape(1, 8), bt1.reshape(1, 8), Wt2.T, bt2.reshape(1, 16),
             Wc1.reshape(1, 8), bc1.reshape(1, 8), Wc2.T, bc2.reshape(1, 16),
             Wm1.reshape(1, 8), bm1.reshape(1, 8), Wm2.T, bm2.reshape(1, 16),
             Wq1.reshape(1, 8), bq1.reshape(1, 8), Wq2.T, bq2.reshape(1, 16)]
    out = _tc_stage3(
        p2, h2, xr2, dinv, b2.reshape(1, 16), alpha2.reshape(1, 1),
        batch.reshape(1, N), scalars, heads, Wf1.T, bf1.reshape(1, 80),
        Wf2.T, bf2.reshape(1, 16))
    return out
